# baseline (speedup 1.0000x reference)
import functools

import jax
import jax.numpy as jnp
from jax import lax
from jax.experimental import pallas as pl
from jax.experimental.pallas import tpu as pltpu
from jax.experimental.pallas import tpu_sc as plsc

ROWS = 1024
V = 8192

_info = plsc.get_sparse_core_info()
NC, NS, L = _info.num_cores, _info.num_subcores, _info.num_lanes
NW = NC * NS
RPW = ROWS // NW

_mesh = plsc.VectorSubcoreMesh(core_axis_name="c", subcore_axis_name="s")


@functools.partial(
    pl.kernel,
    mesh=_mesh,
    out_type=jax.ShapeDtypeStruct((ROWS, L), jnp.float32),
    scratch_types=[
        pltpu.VMEM((V,), jnp.float32),
        pltpu.VMEM((L,), jnp.float32),
    ],
)
def _sc_expsum(x_hbm, out_hbm, rowbuf, esbuf):
    wid = lax.axis_index("s") * NC + lax.axis_index("c")
    base = wid * RPW
    for r in range(RPW):
        pltpu.sync_copy(x_hbm.at[base + r], rowbuf)

        def step(i, carry):
            a, b, c, d = carry
            o = i * (4 * L)
            a = a + jnp.exp(rowbuf[pl.ds(o, L)])
            b = b + jnp.exp(rowbuf[pl.ds(o + L, L)])
            c = c + jnp.exp(rowbuf[pl.ds(o + 2 * L, L)])
            d = d + jnp.exp(rowbuf[pl.ds(o + 3 * L, L)])
            return (a, b, c, d)

        z = jnp.zeros((L,), jnp.float32)
        a, b, c, d = lax.fori_loop(0, V // (4 * L), step, (z, z, z, z))
        esbuf[...] = (a + b) + (c + d)
        pltpu.sync_copy(esbuf, out_hbm.at[base + r])


@jax.jit
def _run(logits, scores):
    x = logits.reshape(ROWS, V)
    es = _sc_expsum(x)
    return (jnp.zeros((4, 32), jnp.float32) + es[0, 0],
            jnp.zeros((4, 32, 8), jnp.int32) + es[0, 0].astype(jnp.int32))


def kernel(logits, scores, beam_size):
    del beam_size
    return _run(logits, scores)


# speedup vs baseline: 2.1884x; 2.1884x over previous
import jax
import jax.numpy as jnp
from jax.experimental import pallas as pl
from jax.experimental.pallas import tpu as pltpu

ROWS = 1024
V = 8192
RSUB = 32
NSUB = 4
RBLK = RSUB * NSUB
NSTEP = ROWS // RBLK
KTOP = 4
NLANE = 128
NCHUNK = V // NLANE
NEG = float("-inf")


RLANE = 8
NILV = RSUB // RLANE


def _cascade_step(state, v, gv):
    t1, t2, t3, t4, g1, g2, g3, g4, es = state
    es = es + jnp.exp(v)
    c1 = v > t1
    nt1 = jnp.maximum(t1, v)
    ng1 = jnp.where(c1, gv, g1)
    cv = jnp.minimum(t1, v)
    cg = jnp.where(c1, g1, gv)
    c2 = cv > t2
    nt2 = jnp.maximum(t2, cv)
    ng2 = jnp.where(c2, cg, g2)
    cv2 = jnp.minimum(t2, cv)
    cg2 = jnp.where(c2, g2, cg)
    c3 = cv2 > t3
    nt3 = jnp.maximum(t3, cv2)
    ng3 = jnp.where(c3, cg2, g3)
    cv3 = jnp.minimum(t3, cv2)
    cg3 = jnp.where(c3, g3, cg2)
    c4 = cv3 > t4
    nt4 = jnp.maximum(t4, cv3)
    ng4 = jnp.where(c4, cg3, g4)
    return (nt1, nt2, nt3, nt4, ng1, ng2, ng3, ng4, es)


def _scan_sub(x_ref, s):
    shape = (RLANE, NLANE)
    init = (jnp.full(shape, NEG),) * 4 + (jnp.zeros(shape, jnp.int32),) * 4 \
        + (jnp.zeros(shape, jnp.float32),)
    states = [init for _ in range(NILV)]
    r0 = s * RSUB
    for i in range(NCHUNK):
        gv = jnp.full(shape, i, jnp.int32)
        for q in range(NILV):
            rq = r0 + q * RLANE
            v = x_ref[rq:rq + RLANE, i * NLANE:(i + 1) * NLANE]
            states[q] = _cascade_step(states[q], v, gv)
    ts = [jnp.concatenate([states[q][j] for q in range(NILV)], axis=0)
          for j in range(4)]
    gs = [jnp.concatenate([states[q][j] for q in range(NILV)], axis=0)
          for j in range(4, 8)]
    es = jnp.concatenate([states[q][8] for q in range(NILV)], axis=0)
    return tuple(ts), tuple(gs), es


def _bulk_phase(scores, ts_s, gs_s, es_s, best_ref, gen_ref):
    full = (ROWS, NLANE)
    ts = [r[...] for r in ts_s]
    lane = jax.lax.broadcasted_iota(jnp.int32, full, 1)
    idxs = [r[...] * NLANE + lane for r in gs_s]
    BIG = jnp.int32(2 * V)

    vals, mis = [], []
    for _ in range(KTOP):
        m4 = jnp.maximum(jnp.maximum(ts[0], ts[1]), jnp.maximum(ts[2], ts[3]))
        rowmax = jnp.max(m4, axis=1, keepdims=True)
        cand = BIG
        eqs = []
        for r in range(KTOP):
            eq = ts[r] == rowmax
            eqs.append(eq)
            cand = jnp.minimum(cand, jnp.where(eq, idxs[r], BIG))
        mi = jnp.min(cand, axis=1, keepdims=True)
        for r in range(KTOP):
            ts[r] = jnp.where(eqs[r] & (idxs[r] == mi), NEG, ts[r])
        vals.append(rowmax)
        mis.append(mi)

    lse = jnp.log(es_s[...])
    adj = jnp.concatenate(vals, axis=1) - lse

    r_io = jax.lax.broadcasted_iota(jnp.int32, (128, ROWS), 1)
    g_io = jax.lax.broadcasted_iota(jnp.int32, (128, ROWS), 0)
    tgt = ((r_io >> 3) & 3) * 32 + (r_io >> 5)
    PT = (g_io == tgt).astype(jnp.float32)

    sums128 = jnp.dot(PT, adj, precision=jax.lax.Precision.HIGHEST,
                      preferred_element_type=jnp.float32)
    i4 = jax.lax.broadcasted_iota(jnp.int32, (4, 4), 0)
    I4 = (i4 == jax.lax.broadcasted_iota(jnp.int32, (4, 4), 1)).astype(jnp.float32)
    scT = jax.lax.dot_general(scores, I4, (((0,), (0,)), ((), ())),
                              precision=jax.lax.Precision.HIGHEST,
                              preferred_element_type=jnp.float32)
    cand_cols = []
    for bm in range(4):
        blk = sums128[bm * RSUB:(bm + 1) * RSUB, :]
        cand_cols.append(blk + scT[:, bm:bm + 1])
    cand = jnp.concatenate(cand_cols, axis=1)

    c_io = jax.lax.broadcasted_iota(jnp.int32, (ROWS, 8), 1)
    rr_io = jax.lax.broadcasted_iota(jnp.int32, (ROWS, 8), 0)
    C8 = ((rr_io & 7) == c_io).astype(jnp.float32)
    Gs = []
    for k in range(KTOP):
        Bk = mis[k].astype(jnp.float32) * C8
        Gs.append(jnp.dot(PT, Bk, precision=jax.lax.Precision.HIGHEST,
                          preferred_element_type=jnp.float32))
    pieces = []
    for bm in range(4):
        for k in range(KTOP):
            pieces.append(Gs[k][bm * RSUB:(bm + 1) * RSUB, :].astype(jnp.int32))

    iota16 = jax.lax.broadcasted_iota(jnp.int32, (RSUB, 16), 1)
    cur = cand
    best_cols = []
    for j in range(KTOP):
        mj = jnp.max(cur, axis=1, keepdims=True)
        eq = cur == mj
        ij = jnp.min(jnp.where(eq, iota16, 16), axis=1, keepdims=True)
        cur = jnp.where(iota16 == ij, NEG, cur)
        best_cols.append(mj)
        acc = jnp.zeros((RSUB, 8), jnp.int32)
        for r in range(16):
            acc = acc + jnp.where(ij == r, pieces[r], 0)
        gen_ref[j] = acc
    best_t = jnp.concatenate(best_cols, axis=1)
    i32 = jax.lax.broadcasted_iota(jnp.int32, (RSUB, RSUB), 0)
    I32 = (i32 == jax.lax.broadcasted_iota(jnp.int32, (RSUB, RSUB), 1)).astype(jnp.float32)
    best_ref[...] = jax.lax.dot_general(
        best_t, I32, (((0,), (0,)), ((), ())),
        precision=jax.lax.Precision.HIGHEST,
        preferred_element_type=jnp.float32)


def _body(x_ref, sc_ref, best_ref, gen_ref,
          t1_s, t2_s, t3_s, t4_s, gg1_s, gg2_s, gg3_s, gg4_s, es_s):
    i = pl.program_id(0)
    for s in range(NSUB):
        (t1, t2, t3, t4), (g1, g2, g3, g4), es = _scan_sub(x_ref, s)
        sl = pl.ds(i * RBLK + s * RSUB, RSUB)
        t1_s[sl, :] = t1
        t2_s[sl, :] = t2
        t3_s[sl, :] = t3
        t4_s[sl, :] = t4
        gg1_s[sl, :] = g1
        gg2_s[sl, :] = g2
        gg3_s[sl, :] = g3
        gg4_s[sl, :] = g4
        es_s[sl, :] = jnp.sum(es, axis=1, keepdims=True)

    @pl.when(i == NSTEP - 1)
    def _():
        _bulk_phase(sc_ref[...], (t1_s, t2_s, t3_s, t4_s),
                    (gg1_s, gg2_s, gg3_s, gg4_s), es_s,
                    best_ref, gen_ref)


@jax.jit
def _run(logits, scores):
    x = logits.reshape(ROWS, V)
    outs = pl.pallas_call(
        _body,
        grid=(NSTEP,),
        in_specs=[
            pl.BlockSpec((RBLK, V), lambda i: (i, 0)),
            pl.BlockSpec((KTOP, RSUB), lambda i: (0, 0)),
        ],
        out_specs=[
            pl.BlockSpec((KTOP, RSUB), lambda i: (0, 0)),
            pl.BlockSpec((KTOP, RSUB, 8), lambda i: (0, 0, 0)),
        ],
        out_shape=[
            jax.ShapeDtypeStruct((KTOP, RSUB), jnp.float32),
            jax.ShapeDtypeStruct((KTOP, RSUB, 8), jnp.int32),
        ],
        scratch_shapes=[
            pltpu.VMEM((ROWS, NLANE), jnp.float32),
            pltpu.VMEM((ROWS, NLANE), jnp.float32),
            pltpu.VMEM((ROWS, NLANE), jnp.float32),
            pltpu.VMEM((ROWS, NLANE), jnp.float32),
            pltpu.VMEM((ROWS, NLANE), jnp.int32),
            pltpu.VMEM((ROWS, NLANE), jnp.int32),
            pltpu.VMEM((ROWS, NLANE), jnp.int32),
            pltpu.VMEM((ROWS, NLANE), jnp.int32),
            pltpu.VMEM((ROWS, 1), jnp.float32),
        ],
    )(x, scores)
    best, gen = outs
    return best, gen


def kernel(logits, scores, beam_size):
    del beam_size
    return _run(logits, scores)


# speedup vs baseline: 2.1914x; 1.0014x over previous
import jax
import jax.numpy as jnp
from jax.experimental import pallas as pl
from jax.experimental.pallas import tpu as pltpu

ROWS = 1024
V = 8192
RSUB = 32
NSUB = 4
RBLK = RSUB * NSUB
NSTEP = ROWS // RBLK
KTOP = 4
NLANE = 128
NCHUNK = V // NLANE
NEG = float("-inf")


RLANE = 8
NILV = RSUB // RLANE


def _cascade_step(state, v, gv):
    t1, t2, t3, t4, g1, g2, g3, g4, es = state
    es = es + jnp.exp(v)
    c1 = v > t1
    nt1 = jnp.maximum(t1, v)
    ng1 = jnp.where(c1, gv, g1)
    cv = jnp.minimum(t1, v)
    cg = jnp.where(c1, g1, gv)
    c2 = cv > t2
    nt2 = jnp.maximum(t2, cv)
    ng2 = jnp.where(c2, cg, g2)
    cv2 = jnp.minimum(t2, cv)
    cg2 = jnp.where(c2, g2, cg)
    c3 = cv2 > t3
    nt3 = jnp.maximum(t3, cv2)
    ng3 = jnp.where(c3, cg2, g3)
    cv3 = jnp.minimum(t3, cv2)
    cg3 = jnp.where(c3, g3, cg2)
    c4 = cv3 > t4
    nt4 = jnp.maximum(t4, cv3)
    ng4 = jnp.where(c4, cg3, g4)
    return (nt1, nt2, nt3, nt4, ng1, ng2, ng3, ng4, es)


def _scan_sub(x_ref, s):
    shape = (RLANE, NLANE)
    init = (jnp.full(shape, NEG),) * 4 + (jnp.zeros(shape, jnp.int32),) * 4 \
        + (jnp.zeros(shape, jnp.float32),)
    states = [init for _ in range(NILV)]
    r0 = s * RSUB
    for i in range(NCHUNK):
        gv = jnp.int32(i)
        for q in range(NILV):
            rq = r0 + q * RLANE
            v = x_ref[rq:rq + RLANE, i * NLANE:(i + 1) * NLANE]
            states[q] = _cascade_step(states[q], v, gv)
    ts = [jnp.concatenate([states[q][j] for q in range(NILV)], axis=0)
          for j in range(4)]
    gs = [jnp.concatenate([states[q][j] for q in range(NILV)], axis=0)
          for j in range(4, 8)]
    es = jnp.concatenate([states[q][8] for q in range(NILV)], axis=0)
    return tuple(ts), tuple(gs), es


def _bulk_phase(scores, ts_s, gs_s, es_s, best_ref, gen_ref):
    full = (ROWS, NLANE)
    ts = [r[...] for r in ts_s]
    lane = jax.lax.broadcasted_iota(jnp.int32, full, 1)
    idxs = [r[...] * NLANE + lane for r in gs_s]
    BIG = jnp.int32(2 * V)

    vals, mis = [], []
    for _ in range(KTOP):
        m4 = jnp.maximum(jnp.maximum(ts[0], ts[1]), jnp.maximum(ts[2], ts[3]))
        rowmax = jnp.max(m4, axis=1, keepdims=True)
        cand = BIG
        eqs = []
        for r in range(KTOP):
            eq = ts[r] == rowmax
            eqs.append(eq)
            cand = jnp.minimum(cand, jnp.where(eq, idxs[r], BIG))
        mi = jnp.min(cand, axis=1, keepdims=True)
        for r in range(KTOP):
            ts[r] = jnp.where(eqs[r] & (idxs[r] == mi), NEG, ts[r])
        vals.append(rowmax)
        mis.append(mi)

    lse = jnp.log(es_s[...])
    adj = jnp.concatenate(vals, axis=1) - lse

    r_io = jax.lax.broadcasted_iota(jnp.int32, (128, ROWS), 1)
    g_io = jax.lax.broadcasted_iota(jnp.int32, (128, ROWS), 0)
    tgt = ((r_io >> 3) & 3) * 32 + (r_io >> 5)
    PT = (g_io == tgt).astype(jnp.float32)

    sums128 = jnp.dot(PT, adj, precision=jax.lax.Precision.HIGHEST,
                      preferred_element_type=jnp.float32)
    i4 = jax.lax.broadcasted_iota(jnp.int32, (4, 4), 0)
    I4 = (i4 == jax.lax.broadcasted_iota(jnp.int32, (4, 4), 1)).astype(jnp.float32)
    scT = jax.lax.dot_general(scores, I4, (((0,), (0,)), ((), ())),
                              precision=jax.lax.Precision.HIGHEST,
                              preferred_element_type=jnp.float32)
    cand_cols = []
    for bm in range(4):
        blk = sums128[bm * RSUB:(bm + 1) * RSUB, :]
        cand_cols.append(blk + scT[:, bm:bm + 1])
    cand = jnp.concatenate(cand_cols, axis=1)

    c_io = jax.lax.broadcasted_iota(jnp.int32, (ROWS, 8), 1)
    rr_io = jax.lax.broadcasted_iota(jnp.int32, (ROWS, 8), 0)
    C8 = ((rr_io & 7) == c_io).astype(jnp.float32)
    Gs = []
    for k in range(KTOP):
        Bk = mis[k].astype(jnp.float32) * C8
        Gs.append(jnp.dot(PT, Bk, precision=jax.lax.Precision.HIGHEST,
                          preferred_element_type=jnp.float32))
    pieces = []
    for bm in range(4):
        for k in range(KTOP):
            pieces.append(Gs[k][bm * RSUB:(bm + 1) * RSUB, :].astype(jnp.int32))

    iota16 = jax.lax.broadcasted_iota(jnp.int32, (RSUB, 16), 1)
    cur = cand
    best_cols = []
    for j in range(KTOP):
        mj = jnp.max(cur, axis=1, keepdims=True)
        eq = cur == mj
        ij = jnp.min(jnp.where(eq, iota16, 16), axis=1, keepdims=True)
        cur = jnp.where(iota16 == ij, NEG, cur)
        best_cols.append(mj)
        acc = jnp.zeros((RSUB, 8), jnp.int32)
        for r in range(16):
            acc = acc + jnp.where(ij == r, pieces[r], 0)
        gen_ref[j] = acc
    best_t = jnp.concatenate(best_cols, axis=1)
    i32 = jax.lax.broadcasted_iota(jnp.int32, (RSUB, RSUB), 0)
    I32 = (i32 == jax.lax.broadcasted_iota(jnp.int32, (RSUB, RSUB), 1)).astype(jnp.float32)
    best_ref[...] = jax.lax.dot_general(
        best_t, I32, (((0,), (0,)), ((), ())),
        precision=jax.lax.Precision.HIGHEST,
        preferred_element_type=jnp.float32)


def _body(x_ref, sc_ref, best_ref, gen_ref,
          t1_s, t2_s, t3_s, t4_s, gg1_s, gg2_s, gg3_s, gg4_s, es_s):
    i = pl.program_id(0)
    for s in range(NSUB):
        (t1, t2, t3, t4), (g1, g2, g3, g4), es = _scan_sub(x_ref, s)
        sl = pl.ds(i * RBLK + s * RSUB, RSUB)
        t1_s[sl, :] = t1
        t2_s[sl, :] = t2
        t3_s[sl, :] = t3
        t4_s[sl, :] = t4
        gg1_s[sl, :] = g1
        gg2_s[sl, :] = g2
        gg3_s[sl, :] = g3
        gg4_s[sl, :] = g4
        es_s[sl, :] = jnp.sum(es, axis=1, keepdims=True)

    @pl.when(i == NSTEP - 1)
    def _():
        _bulk_phase(sc_ref[...], (t1_s, t2_s, t3_s, t4_s),
                    (gg1_s, gg2_s, gg3_s, gg4_s), es_s,
                    best_ref, gen_ref)


@jax.jit
def _run(logits, scores):
    x = logits.reshape(ROWS, V)
    outs = pl.pallas_call(
        _body,
        grid=(NSTEP,),
        in_specs=[
            pl.BlockSpec((RBLK, V), lambda i: (i, 0)),
            pl.BlockSpec((KTOP, RSUB), lambda i: (0, 0)),
        ],
        out_specs=[
            pl.BlockSpec((KTOP, RSUB), lambda i: (0, 0)),
            pl.BlockSpec((KTOP, RSUB, 8), lambda i: (0, 0, 0)),
        ],
        out_shape=[
            jax.ShapeDtypeStruct((KTOP, RSUB), jnp.float32),
            jax.ShapeDtypeStruct((KTOP, RSUB, 8), jnp.int32),
        ],
        scratch_shapes=[
            pltpu.VMEM((ROWS, NLANE), jnp.float32),
            pltpu.VMEM((ROWS, NLANE), jnp.float32),
            pltpu.VMEM((ROWS, NLANE), jnp.float32),
            pltpu.VMEM((ROWS, NLANE), jnp.float32),
            pltpu.VMEM((ROWS, NLANE), jnp.int32),
            pltpu.VMEM((ROWS, NLANE), jnp.int32),
            pltpu.VMEM((ROWS, NLANE), jnp.int32),
            pltpu.VMEM((ROWS, NLANE), jnp.int32),
            pltpu.VMEM((ROWS, 1), jnp.float32),
        ],
    )(x, scores)
    best, gen = outs
    return best, gen


def kernel(logits, scores, beam_size):
    del beam_size
    return _run(logits, scores)
